# Initial kernel scaffold; baseline (speedup 1.0000x reference)
import jax
import jax.numpy as jnp
from jax.experimental import pallas as pl


def kernel(state, x, vw1, vb1, vw2, vb2, sw1, sb1, sw2, sb2):
    raise NotImplementedError("write your pallas kernel here")


# speedup vs baseline: 1.1673x; 1.1673x over previous
import jax
import jax.numpy as jnp
from jax.experimental import pallas as pl
from jax.experimental.pallas import tpu as pltpu

_NSHELF = 64
_SPATIAL = 128
_SLOPE = 0.01
_T = 1024


def _leaky(v):
    return jnp.where(v >= 0, v, _SLOPE * v)


def _body(st_ref, x_ref, vw1_ref, vb1_ref, vw2_ref, vb2_ref,
          sw1_ref, sb1_ref, sw2_ref, sb2_ref, out_ref):
    x = x_ref[...]
    mask = st_ref[...] == _NSHELF
    xv = x[:, :_SPATIAL]
    hv = _leaky(jnp.dot(xv, vw1_ref[...], preferred_element_type=jnp.float32)
                + vb1_ref[...])
    vout = _leaky(jnp.dot(hv, vw2_ref[...], preferred_element_type=jnp.float32)
                  + vb2_ref[...])
    hs = _leaky(jnp.dot(x, sw1_ref[...], preferred_element_type=jnp.float32)
                + sb1_ref[...])
    sout = _leaky(jnp.dot(hs, sw2_ref[...], preferred_element_type=jnp.float32)
                  + sb2_ref[...])
    out_ref[...] = jnp.where(mask, vout, sout)


def kernel(state, x, vw1, vb1, vw2, vb2, sw1, sb1, sw2, sb2):
    B, Nv, F = x.shape
    n_tok = B * Nv
    st = state.reshape(n_tok, 1).astype(jnp.int32)
    xf = x.reshape(n_tok, F)
    grid = (n_tok // _T,)
    full = lambda shape: pl.BlockSpec(shape, lambda i: (0, 0))
    out = pl.pallas_call(
        _body,
        grid=grid,
        in_specs=[
            pl.BlockSpec((_T, 1), lambda i: (i, 0)),
            pl.BlockSpec((_T, F), lambda i: (i, 0)),
            full(vw1.shape), full((1, vb1.shape[0])),
            full(vw2.shape), full((1, vb2.shape[0])),
            full(sw1.shape), full((1, sb1.shape[0])),
            full(sw2.shape), full((1, sb2.shape[0])),
        ],
        out_specs=pl.BlockSpec((_T, 512), lambda i: (i, 0)),
        out_shape=jax.ShapeDtypeStruct((n_tok, 512), jnp.float32),
    )(st, xf, vw1, vb1.reshape(1, -1), vw2, vb2.reshape(1, -1),
      sw1, sb1.reshape(1, -1), sw2, sb2.reshape(1, -1))
    return out.reshape(B, Nv, 512)
